# initial kernel scaffold (unmeasured)
import jax
import jax.numpy as jnp
from jax import lax
from jax.experimental import pallas as pl
from jax.experimental.pallas import tpu as pltpu

L = 8
T_CORR_CHUNKS = 8


def kernel(x, A, B, C):
    B_, SL, D_ = x.shape
    N_ = A.shape[1]
    NC = SL // L

    At = A.T.astype(jnp.float32)
    lvec = jnp.arange(L, dtype=jnp.float32)[:, None, None]
    Eneg = jnp.exp(-At[None] * lvec)
    Escan = jnp.exp(At[None] * lvec)
    dA1 = jnp.exp(At)
    dAL = jnp.exp(At * float(L))

    def body(x_ref, b_ref, c_ref, eneg_ref, escan_ref, da1_ref, dal_ref,
             out_ref, h_ref, h0_ref, send_sem, recv_sem):
        my_x = lax.axis_index("x")
        my_y = lax.axis_index("y")
        peer = (my_x, 1 - my_y)

        barrier = pltpu.get_barrier_semaphore()
        pl.semaphore_signal(barrier, inc=1, device_id=peer,
                            device_id_type=pl.DeviceIdType.MESH)
        pl.semaphore_wait(barrier, 1)

        h_ref[...] = jnp.zeros((B_, N_, D_), dtype=jnp.float32)

        def chunk_step(k, carry):
            t0 = k * L
            xc = x_ref[:, pl.ds(t0, L), :]
            Bc = b_ref[:, pl.ds(t0, L), :]
            Cc = c_ref[:, pl.ds(t0, L), :]
            U = (xc[:, :, None, :] * Bc[:, :, :, None]) * eneg_ref[...][None]
            s = U
            sh = 1
            while sh < L:
                pad = jnp.zeros((B_, sh, N_, D_), dtype=s.dtype)
                s = s + jnp.concatenate([pad, s[:, :L - sh]], axis=1)
                sh *= 2
            hin = da1_ref[...][None] * h_ref[...]
            h_all = escan_ref[...][None] * (hin[:, None] + s)
            yc = jnp.sum(h_all * Cc[:, :, :, None], axis=2)
            out_ref[:, pl.ds(t0, L), :] = yc
            h_ref[...] = h_all[:, L - 1]
            return carry

        lax.fori_loop(0, NC, chunk_step, 0)

        @pl.when(my_y == 0)
        def _():
            rdma = pltpu.make_async_remote_copy(
                src_ref=h_ref, dst_ref=h0_ref,
                send_sem=send_sem, recv_sem=recv_sem,
                device_id=peer, device_id_type=pl.DeviceIdType.MESH,
            )
            rdma.start()
            rdma.wait_send()

        @pl.when(my_y == 1)
        def _():
            rdma = pltpu.make_async_remote_copy(
                src_ref=h_ref, dst_ref=h0_ref,
                send_sem=send_sem, recv_sem=recv_sem,
                device_id=peer, device_id_type=pl.DeviceIdType.MESH,
            )
            rdma.wait_recv()
            h0_ref[...] = da1_ref[...][None] * h0_ref[...]

            def corr_step(k, carry):
                t0 = k * L
                Cc = c_ref[:, pl.ds(t0, L), :]
                g = h0_ref[...]
                hc = escan_ref[...][None] * g[:, None]
                corr = jnp.sum(hc * Cc[:, :, :, None], axis=2)
                out_ref[:, pl.ds(t0, L), :] = out_ref[:, pl.ds(t0, L), :] + corr
                h0_ref[...] = g * dal_ref[...][None]
                return carry

            lax.fori_loop(0, T_CORR_CHUNKS, corr_step, 0)

    return pl.pallas_call(
        body,
        out_shape=jax.ShapeDtypeStruct((B_, SL, D_), jnp.float32),
        in_specs=[pl.BlockSpec(memory_space=pltpu.VMEM)] * 7,
        out_specs=pl.BlockSpec(memory_space=pltpu.VMEM),
        scratch_shapes=[
            pltpu.VMEM((B_, N_, D_), jnp.float32),
            pltpu.VMEM((B_, N_, D_), jnp.float32),
            pltpu.SemaphoreType.DMA,
            pltpu.SemaphoreType.DMA,
        ],
        compiler_params=pltpu.CompilerParams(collective_id=0),
    )(x, B, C, Eneg, Escan, dA1, dAL)


# baseline (device time: 266820 ns/iter reference)
import jax
import jax.numpy as jnp
from jax import lax
from jax.experimental import pallas as pl
from jax.experimental.pallas import tpu as pltpu

L = 8
T_CORR_CHUNKS = 8


def kernel(x, A, B, C):
    B_, SL, D_ = x.shape
    N_ = A.shape[1]
    NC = SL // L

    At = A.T.astype(jnp.float32)
    lvec = jnp.arange(L, dtype=jnp.float32)[:, None, None]
    Eneg = jnp.exp(-At[None] * lvec)
    Escan = jnp.exp(At[None] * lvec)
    dA1 = jnp.exp(At)
    dAL = jnp.exp(At * float(L))

    def body(x_ref, b_ref, c_ref, eneg_ref, escan_ref, da1_ref, dal_ref,
             out_ref, h_ref, h0_ref, send_sem, recv_sem):
        my_x = lax.axis_index("x")
        my_y = lax.axis_index("y")
        peer = (my_x, 1 - my_y)

        barrier = pltpu.get_barrier_semaphore()
        pl.semaphore_signal(barrier, inc=1, device_id=peer,
                            device_id_type=pl.DeviceIdType.MESH)
        pl.semaphore_wait(barrier, 1)

        h_ref[...] = jnp.zeros((B_, N_, D_), dtype=jnp.float32)

        def chunk_step(k, carry):
            t0 = k * L
            xc = x_ref[:, pl.ds(t0, L), :]
            Bc = b_ref[:, pl.ds(t0, L), :]
            Cc = c_ref[:, pl.ds(t0, L), :]
            U = (xc[:, :, None, :] * Bc[:, :, :, None]) * eneg_ref[...][None]
            s = U
            sh = 1
            while sh < L:
                pad = jnp.zeros((B_, sh, N_, D_), dtype=s.dtype)
                s = s + jnp.concatenate([pad, s[:, :L - sh]], axis=1)
                sh *= 2
            hin = da1_ref[...][None] * h_ref[...]
            h_all = escan_ref[...][None] * (hin[:, None] + s)
            yc = jnp.sum(h_all * Cc[:, :, :, None], axis=2)
            out_ref[:, pl.ds(t0, L), :] = yc
            h_ref[...] = h_all[:, L - 1]
            return carry

        lax.fori_loop(0, NC, chunk_step, 0)

        @pl.when(my_y == 0)
        def _():
            rdma = pltpu.make_async_remote_copy(
                src_ref=h_ref, dst_ref=h0_ref,
                send_sem=send_sem, recv_sem=recv_sem,
                device_id=peer, device_id_type=pl.DeviceIdType.MESH,
            )
            rdma.start()
            rdma.wait_send()

        @pl.when(my_y == 1)
        def _():
            rdma = pltpu.make_async_remote_copy(
                src_ref=h_ref, dst_ref=h0_ref,
                send_sem=send_sem, recv_sem=recv_sem,
                device_id=peer, device_id_type=pl.DeviceIdType.MESH,
            )
            rdma.wait_recv()
            h0_ref[...] = da1_ref[...][None] * h0_ref[...]

            def corr_step(k, carry):
                t0 = k * L
                Cc = c_ref[:, pl.ds(t0, L), :]
                g = h0_ref[...]
                hc = escan_ref[...][None] * g[:, None]
                corr = jnp.sum(hc * Cc[:, :, :, None], axis=2)
                out_ref[:, pl.ds(t0, L), :] = out_ref[:, pl.ds(t0, L), :] + corr
                h0_ref[...] = g * dal_ref[...][None]
                return carry

            lax.fori_loop(0, T_CORR_CHUNKS, corr_step, 0)

    return pl.pallas_call(
        body,
        out_shape=jax.ShapeDtypeStruct((B_, SL, D_), jnp.float32),
        in_specs=[pl.BlockSpec(memory_space=pltpu.VMEM)] * 7,
        out_specs=pl.BlockSpec(memory_space=pltpu.VMEM),
        scratch_shapes=[
            pltpu.VMEM((B_, N_, D_), jnp.float32),
            pltpu.VMEM((B_, N_, D_), jnp.float32),
            pltpu.SemaphoreType.DMA,
            pltpu.SemaphoreType.DMA,
        ],
        compiler_params=pltpu.CompilerParams(
            collective_id=0, vmem_limit_bytes=60 * 1024 * 1024
        ),
    )(x, B, C, Eneg, Escan, dA1, dAL)
